# baseline (speedup 1.0000x reference)
import math
import functools
import jax
import jax.numpy as jnp
from jax.experimental import pallas as pl
from jax.experimental.pallas import tpu as pltpu

B = 4; K_DIM = 128; L = 96; S = 2048; KNEI = 16; NHEADS = 4; DH = K_DIM // NHEADS; NL = 2
LL = L * L


def _ln(x, scale, bias, eps=1e-5):
    mu = jnp.mean(x, axis=-1, keepdims=True)
    var = jnp.mean((x - mu) ** 2, axis=-1, keepdims=True)
    return (x - mu) / jnp.sqrt(var + eps) * scale + bias


def _attn_layer(scores, Wq, Wk, Wv, Wo, lis, lib, N_idx, N_mask, lin_idx, S_row_mask):
    Bc, Kc, Lc, _ = scores.shape
    X_flat = scores.transpose(0, 2, 3, 1).reshape(Bc, Lc * Lc, Kc)
    H_S = jnp.take_along_axis(X_flat, lin_idx[..., None], axis=1)
    H_S = _ln(H_S, lis, lib)
    Q = (H_S @ Wq.T).reshape(Bc, S, NHEADS, DH)
    Kv = (H_S @ Wk.T).reshape(Bc, S, NHEADS, DH)
    Vv = (H_S @ Wv.T).reshape(Bc, S, NHEADS, DH)
    idx = jnp.clip(N_idx, 0, S - 1)
    Kn = jax.vmap(lambda xb, ib: xb[ib])(Kv, idx)
    Vn = jax.vmap(lambda xb, ib: xb[ib])(Vv, idx)
    sc = jnp.einsum('bshd,bskhd->bshk', Q, Kn) / math.sqrt(DH)
    sc = jnp.where(N_mask[:, :, None, :], sc, -jnp.inf)
    sc = jnp.where(S_row_mask[:, :, None, None], sc, -jnp.inf)
    w = jax.nn.softmax(sc, axis=-1)
    w = jnp.where(jnp.isnan(w), jnp.zeros_like(w), w)
    o = jnp.einsum('bshk,bskhd->bshd', w, Vn).reshape(Bc, S, Kc)
    o = o @ Wo.T
    src = o * S_row_mask[..., None].astype(o.dtype)
    out_flat = jax.vmap(lambda ib, sb: jnp.zeros((Lc * Lc, Kc), sb.dtype).at[ib].set(sb))(lin_idx, src)
    return out_flat.reshape(Bc, Lc, Lc, Kc).transpose(0, 3, 1, 2)


def _copy_kernel(x_ref, o_ref):
    o_ref[...] = x_ref[...]


def kernel(grid_scores, N_idx, N_mask, id2lr_pad, S_row_mask, Wq, Wk, Wv, Wo, ln_in_scale, ln_in_bias, ln_ch_scale, ln_ch_bias):
    lin_idx = jnp.clip(id2lr_pad[..., 0] * L + id2lr_pad[..., 1], 0, L * L - 1)
    scores = grid_scores
    for i in range(NL):
        attn = _attn_layer(scores, Wq[i], Wk[i], Wv[i], Wo[i], ln_in_scale[i], ln_in_bias[i], N_idx, N_mask, lin_idx, S_row_mask)
        x = (scores + attn).transpose(0, 2, 3, 1)
        x = _ln(x, ln_ch_scale[i], ln_ch_bias[i])
        scores = x.transpose(0, 3, 1, 2)
    out = pl.pallas_call(
        _copy_kernel,
        out_shape=jax.ShapeDtypeStruct(scores.shape, scores.dtype),
    )(scores)
    return out


# speedup vs baseline: 6.2546x; 6.2546x over previous
import functools
import jax
import jax.numpy as jnp
from jax import lax
from jax.experimental import pallas as pl
from jax.experimental.pallas import tpu as pltpu
from jax.experimental.pallas import tpu_sc as plsc

B = 4; K_DIM = 128; L = 96; S = 2048; KNEI = 16; NHEADS = 4; DH = K_DIM // NHEADS; NL = 2
LL = L * L
BS = B * S
GR = B * LL
NW = 32
RPW = BS // NW
SCALE = 1.0 / (DH ** 0.5)
EPS = 1e-5

_MESH = plsc.VectorSubcoreMesh(core_axis_name="c", subcore_axis_name="s")
_SC_PARAMS = pltpu.CompilerParams(needs_layout_passes=False,
                                  use_tc_tiling_on_sc=False)


def _wid():
    return lax.axis_index("c") * 16 + lax.axis_index("s")


def _sc_gather_rows_body(src_hbm, gidx_hbm, out_hbm, idx_v, row_v, sem):
    w = _wid()
    pltpu.sync_copy(gidx_hbm.at[w], idx_v)
    for c in range(2):
        pltpu.async_copy(src_hbm.at[idx_v.at[c]], row_v, sem).wait()
        pltpu.sync_copy(row_v, out_hbm.at[pl.ds(w * RPW + c * 128, 128)])


def _sc_gather_rows(src, gidx, n_src_rows):
    return pl.kernel(
        _sc_gather_rows_body,
        out_type=jax.ShapeDtypeStruct((BS, K_DIM), jnp.float32),
        mesh=_MESH,
        compiler_params=_SC_PARAMS,
        scratch_types=[
            pltpu.VMEM((2, 128), jnp.int32),
            pltpu.VMEM((128, K_DIM), jnp.float32),
            pltpu.SemaphoreType.DMA,
        ],
    )(src, gidx)


def _sc_attn_body(qf, kf, vf, nidx_hbm, wres_hbm, o_hbm, ores_hbm,
                  tab, idxb, scT, qob, obuf, widx_v, rbuf, sem):
    cid = lax.axis_index("c")
    sid = lax.axis_index("s")
    b = 2 * cid + sid // 8
    u = sid % 8
    h = u // 2
    half = u % 2
    qbase0 = b * S + half * 1024
    col0 = 32 * h
    iota = lax.iota(jnp.int32, 16)

    pltpu.sync_copy(nidx_hbm.at[pl.ds(qbase0, 1024)], idxb)

    pltpu.sync_copy(kf.at[pl.ds(b * S, S), pl.ds(col0, DH)], tab)

    def chunk_a(c, _):
        pltpu.sync_copy(qf.at[pl.ds(qbase0 + c * 128, 128), pl.ds(col0, DH)], qob)

        def group_a(g, _):
            qrow = iota + g * 16
            qd = [plsc.load_gather(qob, [qrow, jnp.full((16,), d, jnp.int32)])
                  for d in range(DH)]
            srow = c * 128 + g * 16
            for k in range(KNEI):
                ni = plsc.load_gather(idxb, [iota + srow, jnp.full((16,), k, jnp.int32)])
                acc = qd[0] * plsc.load_gather(tab, [ni, jnp.full((16,), 0, jnp.int32)])
                for d in range(1, DH):
                    acc = acc + qd[d] * plsc.load_gather(tab, [ni, jnp.full((16,), d, jnp.int32)])
                scT[k, pl.ds(srow, 16)] = acc * SCALE
            return 0

        lax.fori_loop(0, 8, group_a, 0)
        return 0

    lax.fori_loop(0, 8, chunk_a, 0)

    pltpu.sync_copy(vf.at[pl.ds(b * S, S), pl.ds(col0, DH)], tab)

    def chunk_b(c, _):
        def group_b(g, _):
            srow = c * 128 + g * 16
            sc = [scT[k, pl.ds(srow, 16)] for k in range(KNEI)]
            m = sc[0]
            for k in range(1, KNEI):
                m = jnp.maximum(m, sc[k])
            ek = [jnp.exp(sc[k] - m) for k in range(KNEI)]
            den = ek[0]
            for k in range(1, KNEI):
                den = den + ek[k]
            inv = 1.0 / den
            od = [jnp.zeros((16,), jnp.float32) for _ in range(DH)]
            for k in range(KNEI):
                ni = plsc.load_gather(idxb, [iota + srow, jnp.full((16,), k, jnp.int32)])
                wk = ek[k] * inv
                for d in range(DH):
                    od[d] = od[d] + wk * plsc.load_gather(tab, [ni, jnp.full((16,), d, jnp.int32)])
            orow = iota + g * 16
            for d in range(DH):
                plsc.store_scatter(obuf, [orow, jnp.full((16,), d, jnp.int32)], od[d])
            return 0

        lax.fori_loop(0, 8, group_b, 0)
        pltpu.sync_copy(obuf, o_hbm.at[pl.ds(qbase0 + c * 128, 128), pl.ds(col0, DH)])
        return 0

    lax.fori_loop(0, 8, chunk_b, 0)

    plsc.subcore_barrier()
    w = _wid()
    pltpu.sync_copy(wres_hbm.at[w], widx_v)
    for c in range(2):
        pltpu.async_copy(o_hbm.at[widx_v.at[c]], rbuf, sem).wait()
        pltpu.sync_copy(rbuf, ores_hbm.at[pl.ds(w * RPW + c * 128, 128)])


def _sc_attn(qf, kf, vf, nidx, wres):
    return pl.kernel(
        _sc_attn_body,
        out_type=(jax.ShapeDtypeStruct((BS, K_DIM), jnp.float32),
                  jax.ShapeDtypeStruct((BS, K_DIM), jnp.float32)),
        mesh=_MESH,
        compiler_params=_SC_PARAMS,
        scratch_types=[
            pltpu.VMEM((S, DH), jnp.float32),
            pltpu.VMEM((1024, KNEI), jnp.int32),
            pltpu.VMEM((KNEI, 1024), jnp.float32),
            pltpu.VMEM((128, DH), jnp.float32),
            pltpu.VMEM((128, DH), jnp.float32),
            pltpu.VMEM((2, 128), jnp.int32),
            pltpu.VMEM((128, K_DIM), jnp.float32),
            pltpu.SemaphoreType.DMA,
        ],
    )(qf, kf, vf, nidx, wres)


def _sc_assemble_body(ugrid, rows_f, sidx_hbm, out_hbm, big_v, idx_v, row_v, sem):
    cid = lax.axis_index("c")
    sid = lax.axis_index("s")
    copy_base = cid * (GR // 2) + sid * (GR // NW)
    for c in range(2):
        off = copy_base + c * 576
        pltpu.sync_copy(ugrid.at[pl.ds(off, 576)], big_v)
        pltpu.sync_copy(big_v, out_hbm.at[pl.ds(off, 576)])
    plsc.subcore_barrier()
    w = cid * 16 + sid
    pltpu.sync_copy(sidx_hbm.at[w], idx_v)
    for c in range(2):
        pltpu.sync_copy(rows_f.at[pl.ds(w * RPW + c * 128, 128)], row_v)
        pltpu.async_copy(row_v, out_hbm.at[idx_v.at[c]], sem).wait()


def _sc_assemble(ugrid, rows_f, sidx):
    return pl.kernel(
        _sc_assemble_body,
        out_type=jax.ShapeDtypeStruct((GR, K_DIM), jnp.float32),
        mesh=_MESH,
        compiler_params=_SC_PARAMS,
        scratch_types=[
            pltpu.VMEM((576, K_DIM), jnp.float32),
            pltpu.VMEM((2, 128), jnp.int32),
            pltpu.VMEM((128, K_DIM), jnp.float32),
            pltpu.SemaphoreType.DMA,
        ],
    )(ugrid, rows_f, sidx)


def _ln_rows(x, s_ref, b_ref):
    mu = jnp.mean(x, axis=-1, keepdims=True)
    var = jnp.mean((x - mu) ** 2, axis=-1, keepdims=True)
    return (x - mu) * lax.rsqrt(var + EPS) * s_ref[...] + b_ref[...]


def _matT(x, w_ref):
    return lax.dot_general(x, w_ref[...], (((1,), (1,)), ((), ())),
                           preferred_element_type=jnp.float32)


def _tc_qkv1_body(h_ref, lis, lib, wq, wk, wv, q_ref, k_ref, v_ref):
    hn = _ln_rows(h_ref[...], lis, lib)
    q_ref[...] = _matT(hn, wq)
    k_ref[...] = _matT(hn, wk)
    v_ref[...] = _matT(hn, wv)


def _tc_upd_qkv_body(prev_ref, ores_ref, wo, lcs, lcb, lis, lib, wq, wk, wv,
                     rows_ref, q_ref, k_ref, v_ref):
    r2 = _ln_rows(prev_ref[...] + _matT(ores_ref[...], wo), lcs, lcb)
    rows_ref[...] = r2
    hn = _ln_rows(r2, lis, lib)
    q_ref[...] = _matT(hn, wq)
    k_ref[...] = _matT(hn, wk)
    v_ref[...] = _matT(hn, wv)


def _tc_upd_body(prev_ref, ores_ref, wo, lcs, lcb, rows_ref):
    rows_ref[...] = _ln_rows(prev_ref[...] + _matT(ores_ref[...], wo), lcs, lcb)


def _tc_gridln_body(x_ref, s1, b1, s2, b2, o_ref):
    o_ref[...] = _ln_rows(_ln_rows(x_ref[...], s1, b1), s2, b2)


def _rows_spec(blk):
    return pl.BlockSpec((blk, K_DIM), lambda i: (i, 0))


def _full_spec(shape):
    return pl.BlockSpec(shape, lambda i: tuple(0 for _ in shape))


_VEC = _full_spec((1, K_DIM))
_WMAT = _full_spec((K_DIM, K_DIM))
_ROWS_T = jax.ShapeDtypeStruct((BS, K_DIM), jnp.float32)


def _tc_qkv1(h, lis, lib, wq, wk, wv):
    return pl.pallas_call(
        _tc_qkv1_body, grid=(8,),
        in_specs=[_rows_spec(1024), _VEC, _VEC, _WMAT, _WMAT, _WMAT],
        out_specs=[_rows_spec(1024)] * 3,
        out_shape=[_ROWS_T] * 3,
    )(h, lis, lib, wq, wk, wv)


def _tc_upd_qkv(prev, ores, wo, lcs, lcb, lis, lib, wq, wk, wv):
    return pl.pallas_call(
        _tc_upd_qkv_body, grid=(8,),
        in_specs=[_rows_spec(1024), _rows_spec(1024), _WMAT, _VEC, _VEC,
                  _VEC, _VEC, _WMAT, _WMAT, _WMAT],
        out_specs=[_rows_spec(1024)] * 4,
        out_shape=[_ROWS_T] * 4,
    )(prev, ores, wo, lcs, lcb, lis, lib, wq, wk, wv)


def _tc_upd(prev, ores, wo, lcs, lcb):
    return pl.pallas_call(
        _tc_upd_body, grid=(8,),
        in_specs=[_rows_spec(1024), _rows_spec(1024), _WMAT, _VEC, _VEC],
        out_specs=_rows_spec(1024),
        out_shape=_ROWS_T,
    )(prev, ores, wo, lcs, lcb)


def _tc_gridln(gf, s1, b1, s2, b2):
    return pl.pallas_call(
        _tc_gridln_body, grid=(9,),
        in_specs=[_rows_spec(4096), _VEC, _VEC, _VEC, _VEC],
        out_specs=_rows_spec(4096),
        out_shape=jax.ShapeDtypeStruct((GR, K_DIM), jnp.float32),
    )(gf, s1, b1, s2, b2)


def kernel(grid_scores, N_idx, N_mask, id2lr_pad, S_row_mask, Wq, Wk, Wv, Wo,
           ln_in_scale, ln_in_bias, ln_ch_scale, ln_ch_bias):
    f32 = jnp.float32
    lin = jnp.clip(id2lr_pad[..., 0] * L + id2lr_pad[..., 1], 0, LL - 1).astype(jnp.int32)
    barange = jnp.broadcast_to(jnp.arange(S, dtype=jnp.int32)[None], (B, S))
    wcell = jnp.zeros((B, LL), jnp.int32).at[jnp.arange(B)[:, None], lin].set(barange)
    w_s = jnp.take_along_axis(wcell, lin, axis=1)
    n32 = jnp.clip(N_idx, 0, S - 1).astype(jnp.int32).reshape(BS, KNEI)
    gidx = (jnp.arange(B, dtype=jnp.int32)[:, None] * LL + lin).reshape(NW, 2, 128)
    wres = (jnp.arange(B, dtype=jnp.int32)[:, None] * S + w_s).reshape(NW, 2, 128)

    gf = grid_scores.reshape(B, K_DIM, LL).transpose(0, 2, 1).reshape(GR, K_DIM)
    lv = [a.reshape(NL, 1, K_DIM).astype(f32)
          for a in (ln_in_scale, ln_in_bias, ln_ch_scale, ln_ch_bias)]
    lis, lib, lcs, lcb = lv

    rows = _sc_gather_rows(gf, gidx, GR)

    q1, k1, v1 = _tc_qkv1(rows, lis[0], lib[0], Wq[0], Wk[0], Wv[0])
    _, ores1 = _sc_attn(q1, k1, v1, n32, wres)
    rows2, q2, k2, v2 = _tc_upd_qkv(rows, ores1, Wo[0], lcs[0], lcb[0],
                                    lis[1], lib[1], Wq[1], Wk[1], Wv[1])
    _, ores2 = _sc_attn(q2, k2, v2, n32, wres)
    rows_f = _tc_upd(rows2, ores2, Wo[1], lcs[1], lcb[1])

    ugrid = _tc_gridln(gf, lcs[0], lcb[0], lcs[1], lcb[1])
    out = _sc_assemble(ugrid, rows_f, gidx)

    return out.reshape(B, LL, K_DIM).transpose(0, 2, 1).reshape(B, K_DIM, L, L)
